# baseline (device time: 37579 ns/iter reference)
import jax
import jax.numpy as jnp
from jax import lax
from jax.experimental import pallas as pl
from jax.experimental.pallas import tpu as pltpu

N_DEV = 16
NQ = 4
NZ = 4
F8 = jnp.float8_e4m3fn
W_PIPE = 4


def kernel(x, w_mat, scale_x, scale_w):
    m_total, k_per = x.shape
    k_total, n = w_mat.shape
    m_blk = m_total // N_DEV
    grp = NZ * m_blk

    def body(x_ref, w_ref, sx_ref, sw_ref, out_ref,
             xsg_ref, p1_ref, p2_ref, p2s_ref, xt_ref, wbuf_ref, wf_ref,
             p1_send, p1_recv, p2_send, p2_recv, w_sems):
        me = lax.axis_index("i")
        q = lax.rem(me, NQ)
        z = lax.rem(lax.div(me, NQ), NZ)

        for b in range(N_DEV):
            q_d, z_d = b // NZ, b % NZ
            xsg_ref[pl.ds(b * m_blk, m_blk), :] = (
                x_ref[pl.ds((NQ * z_d + q_d) * m_blk, m_blk), :].astype(F8))

        p1_descs = []
        for dq in range(1, NQ):
            q_t = lax.rem(q + dq, NQ)
            tgt = me - q + q_t
            d = pltpu.make_async_remote_copy(
                src_ref=xsg_ref.at[pl.ds(q_t * grp, grp), :],
                dst_ref=p1_ref.at[dq],
                send_sem=p1_send.at[dq],
                recv_sem=p1_recv.at[dq],
                device_id=(tgt,),
                device_id_type=pl.DeviceIdType.MESH,
            )
            d.start()
            p1_descs.append(d)

        p1_ref[0, :, :] = xsg_ref[pl.ds(q * grp, grp), :]

        def src_of(c):
            dz, s = c // NQ, c % NQ
            z_s = lax.rem(z - dz + NZ, NZ)
            q_s = lax.rem(q - s + NQ, NQ)
            return NQ * z_s + q_s

        def w_copy(c, slot):
            return pltpu.make_async_copy(
                w_ref.at[pl.ds(src_of(c) * k_per, k_per), :],
                wbuf_ref.at[slot],
                w_sems.at[slot],
            )

        for c in range(W_PIPE):
            w_copy(c, c).start()

        for d in p1_descs:
            d.wait_recv()

        p2_descs = []
        for dz in range(1, NZ):
            z_t = lax.rem(z + dz, NZ)
            tgt = lax.rem(me + NQ * dz, N_DEV)
            for s in range(NQ):
                p2s_ref[dz, pl.ds(s * m_blk, m_blk), :] = (
                    p1_ref[s, pl.ds(z_t * m_blk, m_blk), :])
            d = pltpu.make_async_remote_copy(
                src_ref=p2s_ref.at[dz],
                dst_ref=p2_ref.at[dz],
                send_sem=p2_send.at[dz],
                recv_sem=p2_recv.at[dz],
                device_id=(tgt,),
                device_id_type=pl.DeviceIdType.MESH,
            )
            d.start()
            p2_descs.append(d)

        for c in range(N_DEV):
            w_copy(c, c % W_PIPE).wait()
            wf_ref[pl.ds(c * k_per, k_per), :] = wbuf_ref[c % W_PIPE].astype(F8)
            if c + W_PIPE < N_DEV:
                w_copy(c + W_PIPE, c % W_PIPE).start()

        for s in range(NQ):
            xt_ref[:, pl.ds(s * k_per, k_per)] = (
                p1_ref[s, pl.ds(z * m_blk, m_blk), :])
        for i, d in enumerate(p2_descs):
            dz = i + 1
            d.wait_recv()
            for s in range(NQ):
                xt_ref[:, pl.ds((dz * NQ + s) * k_per, k_per)] = (
                    p2_ref[dz, pl.ds(s * m_blk, m_blk), :])

        acc = lax.dot_general(
            xt_ref[...], wf_ref[...], (((1,), (0,)), ((), ())),
            preferred_element_type=jnp.float32)
        sc = sx_ref[0, 0] * sw_ref[0, 0]
        out_ref[...] = jnp.maximum(acc * sc, 0.0)

        for d in p1_descs + p2_descs:
            d.wait_send()

    return pl.pallas_call(
        body,
        out_shape=jax.ShapeDtypeStruct((m_blk, n), jnp.float32),
        in_specs=[
            pl.BlockSpec(memory_space=pltpu.VMEM),
            pl.BlockSpec(memory_space=pltpu.MemorySpace.HBM),
            pl.BlockSpec(memory_space=pltpu.SMEM),
            pl.BlockSpec(memory_space=pltpu.SMEM),
        ],
        out_specs=pl.BlockSpec(memory_space=pltpu.VMEM),
        scratch_shapes=[
            pltpu.VMEM((m_total, k_per), F8),
            pltpu.VMEM((NQ, NZ * m_blk, k_per), F8),
            pltpu.VMEM((NZ, NQ * m_blk, k_per), F8),
            pltpu.VMEM((NZ, NQ * m_blk, k_per), F8),
            pltpu.VMEM((m_blk, k_total), F8),
            pltpu.VMEM((W_PIPE, k_per, n), jnp.float32),
            pltpu.VMEM((k_total, n), F8),
            pltpu.SemaphoreType.DMA((NQ,)),
            pltpu.SemaphoreType.DMA((NQ,)),
            pltpu.SemaphoreType.DMA((NZ,)),
            pltpu.SemaphoreType.DMA((NZ,)),
            pltpu.SemaphoreType.DMA((W_PIPE,)),
        ],
        compiler_params=pltpu.CompilerParams(
            vmem_limit_bytes=100 * 1024 * 1024,
        ),
    )(x, w_mat, scale_x.reshape(1, 1), scale_w.reshape(1, 1))


# device time: 29181 ns/iter; 1.2878x vs baseline; 1.2878x over previous
import jax
import jax.numpy as jnp
from jax import lax
from jax.experimental import pallas as pl
from jax.experimental.pallas import tpu as pltpu

N_DEV = 16
F8 = jnp.float8_e4m3fn
W_PIPE = 4


def kernel(x, w_mat, scale_x, scale_w):
    m_total, k_per = x.shape
    k_total, n = w_mat.shape
    m_blk = m_total // N_DEV

    def body(x_ref, w_ref, sx_ref, sw_ref, out_ref,
             xs_ref, comm_ref, xt_ref, wbuf_ref, wf_ref,
             send_sems, recv_sems, w_sems):
        me = lax.axis_index("i")

        xs_ref[...] = x_ref[...].astype(F8)

        send_descs = []
        for off in range(1, N_DEV):
            tgt = lax.rem(me + off, N_DEV)
            d = pltpu.make_async_remote_copy(
                src_ref=xs_ref.at[pl.ds(tgt * m_blk, m_blk), :],
                dst_ref=comm_ref.at[off],
                send_sem=send_sems.at[off],
                recv_sem=recv_sems.at[off],
                device_id=(tgt,),
                device_id_type=pl.DeviceIdType.MESH,
            )
            d.start()
            send_descs.append(d)

        def w_copy(off, slot):
            src = lax.rem(me - off + N_DEV, N_DEV)
            return pltpu.make_async_copy(
                w_ref.at[pl.ds(src * k_per, k_per), :],
                wbuf_ref.at[slot],
                w_sems.at[slot],
            )

        for c in range(W_PIPE):
            w_copy(c, c).start()

        for c in range(N_DEV):
            w_copy(c, c % W_PIPE).wait()
            wf_ref[pl.ds(c * k_per, k_per), :] = wbuf_ref[c % W_PIPE].astype(F8)
            if c + W_PIPE < N_DEV:
                w_copy(c + W_PIPE, c % W_PIPE).start()

        xt_ref[:, pl.ds(0, k_per)] = xs_ref[pl.ds(me * m_blk, m_blk), :]
        for off in range(1, N_DEV):
            send_descs[off - 1].wait_recv()
            xt_ref[:, pl.ds(off * k_per, k_per)] = comm_ref[off]

        acc = lax.dot_general(
            xt_ref[...], wf_ref[...], (((1,), (0,)), ((), ())),
            preferred_element_type=jnp.float32)
        s = sx_ref[0, 0] * sw_ref[0, 0]
        out_ref[...] = jnp.maximum(acc * s, 0.0)

        for d in send_descs:
            d.wait_send()

    return pl.pallas_call(
        body,
        out_shape=jax.ShapeDtypeStruct((m_blk, n), jnp.float32),
        in_specs=[
            pl.BlockSpec(memory_space=pltpu.VMEM),
            pl.BlockSpec(memory_space=pltpu.MemorySpace.HBM),
            pl.BlockSpec(memory_space=pltpu.SMEM),
            pl.BlockSpec(memory_space=pltpu.SMEM),
        ],
        out_specs=pl.BlockSpec(memory_space=pltpu.VMEM),
        scratch_shapes=[
            pltpu.VMEM((m_total, k_per), F8),
            pltpu.VMEM((N_DEV, m_blk, k_per), F8),
            pltpu.VMEM((m_blk, k_total), F8),
            pltpu.VMEM((W_PIPE, k_per, n), jnp.float32),
            pltpu.VMEM((k_total, n), F8),
            pltpu.SemaphoreType.DMA((N_DEV,)),
            pltpu.SemaphoreType.DMA((N_DEV,)),
            pltpu.SemaphoreType.DMA((W_PIPE,)),
        ],
        compiler_params=pltpu.CompilerParams(
            vmem_limit_bytes=100 * 1024 * 1024,
        ),
    )(x, w_mat, scale_x.reshape(1, 1), scale_w.reshape(1, 1))
